# baseline (speedup 1.0000x reference)
import functools

import jax
import jax.numpy as jnp
from jax import lax
from jax.experimental import pallas as pl
from jax.experimental.pallas import tpu as pltpu
from jax.experimental.pallas import tpu_sc as plsc

N = 10000
E = 320000
D = 128

NC = 2
NS = 16

EPT = E // NS
CH = 88
NCHUNK = -(-EPT // CH)
EPT_P = NCHUNK * CH
SEC = NS * EPT_P
NBUF = 4
NGRP = -(-NCHUNK // NBUF)

NP = N + 8

RB = 80
NRC = N // RB
RC_PER_TILE = -(-NRC // NS)

RO = 80
NOC = N // RO
OC_PER_TILE = -(-NOC // NS)

_mesh = plsc.VectorSubcoreMesh(
    core_axis_name="c", subcore_axis_name="s", num_cores=NC, num_subcores=NS)


@functools.partial(
    pl.kernel,
    out_type=jax.ShapeDtypeStruct((N, 2 * D), jnp.float32),
    mesh=_mesh,
    scratch_types=[
        pltpu.VMEM_SHARED((NP, D), jnp.float32),
        [pltpu.VMEM((2, CH), jnp.int32)] * NBUF,
        [pltpu.VMEM((CH, D), jnp.float32)] * NBUF,
        [pltpu.SemaphoreType.DMA] * NBUF,
        [pltpu.SemaphoreType.DMA] * NBUF,
        [pltpu.SemaphoreType.DMA] * NBUF,
        pltpu.SemaphoreType.DMA,
    ],
)
def _h2gcn_sc(x_hbm, edges_hbm, out_hbm, acc, idxb, rows, isem, gsem, ssem,
              zsem):
    c = lax.axis_index("c")
    s = lax.axis_index("s")

    zvec = jnp.zeros((16,), jnp.float32)

    def zfill(i, _):
        rows[0][i // 8, pl.ds((i % 8) * 16, 16)] = zvec
        return 0
    lax.fori_loop(0, RB * D // 16, zfill, 0)

    tb = s * EPT_P
    src_off = (2 * c) * SEC + tb
    dst_off = (2 * c + 1) * SEC + tb

    def fetch(j, b):
        off = j * CH
        pltpu.async_copy(edges_hbm.at[pl.ds(src_off + off, CH)],
                         idxb[b].at[0], isem[b])
        pltpu.async_copy(edges_hbm.at[pl.ds(dst_off + off, CH)],
                         idxb[b].at[1], isem[b])

    def fetch_wait(b):
        pltpu.make_async_copy(edges_hbm.at[pl.ds(src_off, CH)],
                              idxb[b].at[0], isem[b]).wait()
        pltpu.make_async_copy(edges_hbm.at[pl.ds(dst_off, CH)],
                              idxb[b].at[1], isem[b]).wait()

    for b in range(NBUF):
        fetch(b, b)

    def zissue(i, _):
        k = i * NS + s

        @pl.when(k < NRC)
        def _():
            pltpu.async_copy(rows[0].at[pl.ds(0, RB)],
                             acc.at[pl.ds(k * RB, RB)], zsem)
        return 0
    lax.fori_loop(0, RC_PER_TILE, zissue, 0)

    def zwait(i, _):
        k = i * NS + s

        @pl.when(k < NRC)
        def _():
            pltpu.make_async_copy(rows[0].at[pl.ds(0, RB)],
                                  acc.at[pl.ds(0, RB)], zsem).wait()
        return 0
    lax.fori_loop(0, RC_PER_TILE, zwait, 0)

    for b in range(NBUF):
        fetch_wait(b)
        pltpu.async_copy(x_hbm.at[idxb[b].at[0]], rows[b], gsem[b])
    plsc.subcore_barrier()

    @pl.loop(0, NGRP)
    def _(g):
        base = g * NBUF
        for b in range(NBUF):
            j = base + b

            @pl.when(j < NCHUNK)
            def _():
                pltpu.make_async_copy(x_hbm.at[idxb[b].at[0]], rows[b],
                                      gsem[b]).wait()
                pltpu.async_copy(rows[b], acc.at[idxb[b].at[1]], ssem[b],
                                 add=True)
        for b in range(NBUF):
            j = base + b
            j2 = j + NBUF

            @pl.when(j < NCHUNK)
            def _():
                pltpu.make_async_copy(rows[b], acc.at[idxb[b].at[1]],
                                      ssem[b]).wait()

            @pl.when(j2 < NCHUNK)
            def _():
                fetch(j2, b)
                fetch_wait(b)
                pltpu.async_copy(x_hbm.at[idxb[b].at[0]], rows[b], gsem[b])

    plsc.subcore_barrier()

    col = pl.multiple_of(c * D, D)

    def oissue(i, _):
        k = i * NS + s

        @pl.when(k < NOC)
        def _():
            base = k * RO
            pltpu.async_copy(acc.at[pl.ds(base, RO)],
                             out_hbm.at[pl.ds(base, RO), pl.ds(col, D)], zsem)
        return 0
    lax.fori_loop(0, OC_PER_TILE, oissue, 0)

    def owait(i, _):
        k = i * NS + s

        @pl.when(k < NOC)
        def _():
            pltpu.make_async_copy(
                acc.at[pl.ds(0, RO)],
                out_hbm.at[pl.ds(0, RO), pl.ds(col, D)], zsem).wait()
        return 0
    lax.fori_loop(0, OC_PER_TILE, owait, 0)


def _pad_section(row, fill):
    seg = row.reshape(NS, EPT)
    seg = jnp.pad(seg, ((0, 0), (0, EPT_P - EPT)), constant_values=fill)
    return seg.reshape(-1)


def kernel(x, edge_index, edge_index2):
    ei1 = edge_index.astype(jnp.int32)
    ei2 = edge_index2.astype(jnp.int32)
    edges = jnp.concatenate([
        _pad_section(ei1[0], 0), _pad_section(ei1[1], N),
        _pad_section(ei2[0], 0), _pad_section(ei2[1], N)])
    return _h2gcn_sc(x, edges)


# speedup vs baseline: 1.3302x; 1.3302x over previous
import functools

import jax
import jax.numpy as jnp
from jax import lax
from jax.experimental import pallas as pl
from jax.experimental.pallas import tpu as pltpu
from jax.experimental.pallas import tpu_sc as plsc

N = 10000
E = 320000
D = 128

NC = 2
NS = 16

EPT = E // NS
CH = 80
NCHUNK = -(-EPT // CH)
EPT_P = NCHUNK * CH
SEC = NS * EPT_P
NBUF = 4
NGRP = -(-NCHUNK // NBUF)

NP = N + 8

RB = 80
NRC = N // RB
RC_PER_TILE = -(-NRC // NS)

RO = 80
NOC = N // RO
OC_PER_TILE = -(-NOC // NS)

_mesh = plsc.VectorSubcoreMesh(
    core_axis_name="c", subcore_axis_name="s", num_cores=NC, num_subcores=NS)


@functools.partial(
    pl.kernel,
    out_type=jax.ShapeDtypeStruct((N, 2 * D), jnp.float32),
    mesh=_mesh,
    scratch_types=[
        pltpu.VMEM_SHARED((NP, D), jnp.float32),
        [pltpu.VMEM((2, CH), jnp.int32)] * NBUF,
        [pltpu.VMEM((CH, D), jnp.float32)] * NBUF,
        [pltpu.SemaphoreType.DMA] * NBUF,
        [pltpu.SemaphoreType.DMA] * NBUF,
        [pltpu.SemaphoreType.DMA] * NBUF,
        pltpu.SemaphoreType.DMA,
    ],
)
def _h2gcn_sc(x_hbm, edges_hbm, out_hbm, acc, idxb, rows, isem, gsem, ssem,
              zsem):
    c = lax.axis_index("c")
    s = lax.axis_index("s")

    zvec = jnp.zeros((16,), jnp.float32)

    def zfill(i, _):
        rows[0][i // 8, pl.ds((i % 8) * 16, 16)] = zvec
        return 0
    lax.fori_loop(0, RB * D // 16, zfill, 0)

    tb = s * EPT_P
    src_off = (2 * c) * SEC + tb
    dst_off = (2 * c + 1) * SEC + tb

    def fetch(j, b):
        off = j * CH
        pltpu.async_copy(edges_hbm.at[pl.ds(src_off + off, CH)],
                         idxb[b].at[0], isem[b])
        pltpu.async_copy(edges_hbm.at[pl.ds(dst_off + off, CH)],
                         idxb[b].at[1], isem[b])

    def fetch_wait(b):
        pltpu.make_async_copy(edges_hbm.at[pl.ds(src_off, CH)],
                              idxb[b].at[0], isem[b]).wait()
        pltpu.make_async_copy(edges_hbm.at[pl.ds(dst_off, CH)],
                              idxb[b].at[1], isem[b]).wait()

    for b in range(NBUF):
        fetch(b, b)

    def zissue(i, _):
        k = i * NS + s

        @pl.when(k < NRC)
        def _():
            pltpu.async_copy(rows[0].at[pl.ds(0, RB)],
                             acc.at[pl.ds(k * RB, RB)], zsem)
        return 0
    lax.fori_loop(0, RC_PER_TILE, zissue, 0)

    def zwait(i, _):
        k = i * NS + s

        @pl.when(k < NRC)
        def _():
            pltpu.make_async_copy(rows[0].at[pl.ds(0, RB)],
                                  acc.at[pl.ds(0, RB)], zsem).wait()
        return 0
    lax.fori_loop(0, RC_PER_TILE, zwait, 0)

    for b in range(NBUF):
        fetch_wait(b)
        pltpu.async_copy(x_hbm.at[idxb[b].at[0]], rows[b], gsem[b])
    plsc.subcore_barrier()

    @pl.loop(0, NGRP)
    def _(g):
        base = g * NBUF
        for b in range(NBUF):
            j = base + b

            @pl.when(j < NCHUNK)
            def _():
                pltpu.make_async_copy(x_hbm.at[idxb[b].at[0]], rows[b],
                                      gsem[b]).wait()
                pltpu.async_copy(rows[b], acc.at[idxb[b].at[1]], ssem[b],
                                 add=True)
        for b in range(NBUF):
            j = base + b
            j2 = j + NBUF

            @pl.when(j < NCHUNK)
            def _():
                pltpu.make_async_copy(rows[b], acc.at[idxb[b].at[1]],
                                      ssem[b]).wait()

            @pl.when(j2 < NCHUNK)
            def _():
                fetch(j2, b)
                fetch_wait(b)
                pltpu.async_copy(x_hbm.at[idxb[b].at[0]], rows[b], gsem[b])

    plsc.subcore_barrier()

    col = pl.multiple_of(c * D, D)

    def oissue(i, _):
        k = i * NS + s

        @pl.when(k < NOC)
        def _():
            base = k * RO
            pltpu.async_copy(acc.at[pl.ds(base, RO)],
                             out_hbm.at[pl.ds(base, RO), pl.ds(col, D)], zsem)
        return 0
    lax.fori_loop(0, OC_PER_TILE, oissue, 0)

    def owait(i, _):
        k = i * NS + s

        @pl.when(k < NOC)
        def _():
            pltpu.make_async_copy(
                acc.at[pl.ds(0, RO)],
                out_hbm.at[pl.ds(0, RO), pl.ds(col, D)], zsem).wait()
        return 0
    lax.fori_loop(0, OC_PER_TILE, owait, 0)


def _pad_section(row, fill):
    seg = row.reshape(NS, EPT)
    seg = jnp.pad(seg, ((0, 0), (0, EPT_P - EPT)), constant_values=fill)
    return seg.reshape(-1)


def kernel(x, edge_index, edge_index2):
    ei1 = edge_index.astype(jnp.int32)
    ei2 = edge_index2.astype(jnp.int32)
    edges = jnp.concatenate([
        _pad_section(ei1[0], 0), _pad_section(ei1[1], N),
        _pad_section(ei2[0], 0), _pad_section(ei2[1], N)])
    return _h2gcn_sc(x, edges)


# speedup vs baseline: 1.4108x; 1.0606x over previous
import functools

import jax
import jax.numpy as jnp
from jax import lax
from jax.experimental import pallas as pl
from jax.experimental.pallas import tpu as pltpu
from jax.experimental.pallas import tpu_sc as plsc

N = 10000
E = 320000
D = 128

NC = 2
NS = 16

EPT = E // NS
CH = 80
NCHUNK = -(-EPT // CH)
EPT_P = NCHUNK * CH
SEC = NS * EPT_P
NBUF = 4
NIB = 2 * NBUF
NGRP = -(-NCHUNK // NIB)

NP = N + 8

RB = 80
NRC = N // RB
RC_PER_TILE = -(-NRC // NS)

RO = 80
NOC = N // RO
OC_PER_TILE = -(-NOC // NS)

_mesh = plsc.VectorSubcoreMesh(
    core_axis_name="c", subcore_axis_name="s", num_cores=NC, num_subcores=NS)


@functools.partial(
    pl.kernel,
    out_type=jax.ShapeDtypeStruct((N, 2 * D), jnp.float32),
    mesh=_mesh,
    scratch_types=[
        pltpu.VMEM_SHARED((NP, D), jnp.float32),
        [pltpu.VMEM((2, CH), jnp.int32)] * NIB,
        [pltpu.VMEM((CH, D), jnp.float32)] * NBUF,
        [pltpu.SemaphoreType.DMA] * NIB,
        [pltpu.SemaphoreType.DMA] * NBUF,
        [pltpu.SemaphoreType.DMA] * NBUF,
        pltpu.SemaphoreType.DMA,
    ],
)
def _h2gcn_sc(x_hbm, edges_hbm, out_hbm, acc, idxb, rows, isem, gsem, ssem,
              zsem):
    c = lax.axis_index("c")
    s = lax.axis_index("s")

    zvec = jnp.zeros((16,), jnp.float32)

    def zfill(i, _):
        rows[0][i // 8, pl.ds((i % 8) * 16, 16)] = zvec
        return 0
    lax.fori_loop(0, RB * D // 16, zfill, 0)

    tb = s * EPT_P
    src_off = (2 * c) * SEC + tb
    dst_off = (2 * c + 1) * SEC + tb

    def fetch(j, b):
        off = j * CH
        pltpu.async_copy(edges_hbm.at[pl.ds(src_off + off, CH)],
                         idxb[b].at[0], isem[b])
        pltpu.async_copy(edges_hbm.at[pl.ds(dst_off + off, CH)],
                         idxb[b].at[1], isem[b])

    def fetch_wait(b):
        pltpu.make_async_copy(edges_hbm.at[pl.ds(src_off, CH)],
                              idxb[b].at[0], isem[b]).wait()
        pltpu.make_async_copy(edges_hbm.at[pl.ds(dst_off, CH)],
                              idxb[b].at[1], isem[b]).wait()

    for m in range(NIB):
        fetch(m, m)

    def zissue(i, _):
        k = i * NS + s

        @pl.when(k < NRC)
        def _():
            pltpu.async_copy(rows[0].at[pl.ds(0, RB)],
                             acc.at[pl.ds(k * RB, RB)], zsem)
        return 0
    lax.fori_loop(0, RC_PER_TILE, zissue, 0)

    def zwait(i, _):
        k = i * NS + s

        @pl.when(k < NRC)
        def _():
            pltpu.make_async_copy(rows[0].at[pl.ds(0, RB)],
                                  acc.at[pl.ds(0, RB)], zsem).wait()
        return 0
    lax.fori_loop(0, RC_PER_TILE, zwait, 0)

    for b in range(NBUF):
        fetch_wait(b)
        pltpu.async_copy(x_hbm.at[idxb[b].at[0]], rows[b], gsem[b])
    plsc.subcore_barrier()

    @pl.loop(0, NGRP)
    def _(g):
        base = g * NIB
        for h in range(2):
            for b in range(NBUF):
                j = base + h * NBUF + b
                m = h * NBUF + b

                @pl.when(j < NCHUNK)
                def _():
                    pltpu.make_async_copy(x_hbm.at[idxb[m].at[0]], rows[b],
                                          gsem[b]).wait()
                    pltpu.async_copy(rows[b], acc.at[idxb[m].at[1]], ssem[b],
                                     add=True)
            for b in range(NBUF):
                j = base + h * NBUF + b
                m = h * NBUF + b
                jf = j + NIB
                jg = j + NBUF
                mg = (1 - h) * NBUF + b

                @pl.when(j < NCHUNK)
                def _():
                    pltpu.make_async_copy(rows[b], acc.at[idxb[m].at[1]],
                                          ssem[b]).wait()

                @pl.when(jf < NCHUNK)
                def _():
                    fetch(jf, m)

                @pl.when(jg < NCHUNK)
                def _():
                    fetch_wait(mg)
                    pltpu.async_copy(x_hbm.at[idxb[mg].at[0]], rows[b],
                                     gsem[b])

    plsc.subcore_barrier()

    col = pl.multiple_of(c * D, D)

    def oissue(i, _):
        k = i * NS + s

        @pl.when(k < NOC)
        def _():
            base = k * RO
            pltpu.async_copy(acc.at[pl.ds(base, RO)],
                             out_hbm.at[pl.ds(base, RO), pl.ds(col, D)], zsem)
        return 0
    lax.fori_loop(0, OC_PER_TILE, oissue, 0)

    def owait(i, _):
        k = i * NS + s

        @pl.when(k < NOC)
        def _():
            pltpu.make_async_copy(
                acc.at[pl.ds(0, RO)],
                out_hbm.at[pl.ds(0, RO), pl.ds(col, D)], zsem).wait()
        return 0
    lax.fori_loop(0, OC_PER_TILE, owait, 0)


def _pad_section(row, fill):
    seg = row.reshape(NS, EPT)
    seg = jnp.pad(seg, ((0, 0), (0, EPT_P - EPT)), constant_values=fill)
    return seg.reshape(-1)


def kernel(x, edge_index, edge_index2):
    ei1 = edge_index.astype(jnp.int32)
    ei2 = edge_index2.astype(jnp.int32)
    edges = jnp.concatenate([
        _pad_section(ei1[0], 0), _pad_section(ei1[1], N),
        _pad_section(ei2[0], 0), _pad_section(ei2[1], N)])
    return _h2gcn_sc(x, edges)
